# Initial kernel scaffold; baseline (speedup 1.0000x reference)
import jax
import jax.numpy as jnp
from jax.experimental import pallas as pl


def kernel(adj_mtx):
    raise NotImplementedError("write your pallas kernel here")


# speedup vs baseline: 1.5651x; 1.5651x over previous
import functools

import jax
import jax.numpy as jnp
from jax.experimental import pallas as pl


BLK = 512


def _square_kernel(a_ref, m_ref, mb_ref, rs_ref, cs_ref):
    i = pl.program_id(0)
    a = a_ref[...]
    sq = a * a
    m_ref[...] = sq
    mb_ref[...] = sq.astype(jnp.bfloat16)
    rs_ref[...] = jnp.sum(sq, axis=1, keepdims=True)
    part = jnp.sum(sq, axis=0, keepdims=True)

    @pl.when(i == 0)
    def _init():
        cs_ref[...] = part

    @pl.when(i != 0)
    def _acc():
        cs_ref[...] += part


def _dual_matvec_kernel(mb_ref, v_ref, vt_ref, y_ref, yt_ref):
    i = pl.program_id(0)
    m = mb_ref[...].astype(jnp.float32)
    vrow = v_ref[...]
    vtblk = vt_ref[...]
    y_ref[...] = jnp.sum(m * vrow, axis=1, keepdims=True)
    part = jnp.sum(m * vtblk, axis=0, keepdims=True)

    @pl.when(i == 0)
    def _init():
        yt_ref[...] = part

    @pl.when(i != 0)
    def _acc():
        yt_ref[...] += part


def _grad_kernel(vts_ref, v_ref, g_ref):
    i = pl.program_id(0)
    blk, d = g_ref.shape
    g = vts_ref[...] * v_ref[...]
    rows = jax.lax.broadcasted_iota(jnp.int32, (blk, d), 0) + i * blk
    cols = jax.lax.broadcasted_iota(jnp.int32, (blk, d), 1)
    g_ref[...] = g + jnp.where(rows == cols, jnp.float32(100.0), jnp.float32(0.0))


@functools.partial(jax.jit, static_argnums=())
def kernel(adj_mtx):
    d = adj_mtx.shape[0]
    nblk = d // BLK
    f32 = jnp.float32

    matrix, mbf16, rowsum, colsum = pl.pallas_call(
        _square_kernel,
        grid=(nblk,),
        in_specs=[pl.BlockSpec((BLK, d), lambda i: (i, 0))],
        out_specs=[
            pl.BlockSpec((BLK, d), lambda i: (i, 0)),
            pl.BlockSpec((BLK, d), lambda i: (i, 0)),
            pl.BlockSpec((BLK, 1), lambda i: (i, 0)),
            pl.BlockSpec((1, d), lambda i: (0, 0)),
        ],
        out_shape=[
            jax.ShapeDtypeStruct((d, d), f32),
            jax.ShapeDtypeStruct((d, d), jnp.bfloat16),
            jax.ShapeDtypeStruct((d, 1), f32),
            jax.ShapeDtypeStruct((1, d), f32),
        ],
    )(adj_mtx)

    dual_matvec = pl.pallas_call(
        _dual_matvec_kernel,
        grid=(nblk,),
        in_specs=[
            pl.BlockSpec((BLK, d), lambda i: (i, 0)),
            pl.BlockSpec((1, d), lambda i: (0, 0)),
            pl.BlockSpec((BLK, 1), lambda i: (i, 0)),
        ],
        out_specs=[
            pl.BlockSpec((BLK, 1), lambda i: (i, 0)),
            pl.BlockSpec((1, d), lambda i: (0, 0)),
        ],
        out_shape=[
            jax.ShapeDtypeStruct((d, 1), f32),
            jax.ShapeDtypeStruct((1, d), f32),
        ],
    )

    def norm_rows(x):
        return x / jnp.sqrt(jnp.sum(x * x))

    eps_d = jnp.float32(1e-6) * d
    v = norm_rows(rowsum.reshape(1, d) + eps_d)
    vt = norm_rows(colsum + eps_d)

    for _ in range(4):
        y, yt = dual_matvec(mbf16, v, vt.reshape(d, 1))
        v = norm_rows(y.reshape(1, d) + jnp.float32(1e-6) * jnp.sum(v))
        vt = norm_rows(yt + jnp.float32(1e-6) * jnp.sum(vt))

    inv_dot = jnp.float32(1.0) / jnp.sum(v * vt)
    vts = (vt * inv_dot).reshape(d, 1)

    gradient = pl.pallas_call(
        _grad_kernel,
        grid=(nblk,),
        in_specs=[
            pl.BlockSpec((BLK, 1), lambda i: (i, 0)),
            pl.BlockSpec((1, d), lambda i: (0, 0)),
        ],
        out_specs=pl.BlockSpec((BLK, d), lambda i: (i, 0)),
        out_shape=jax.ShapeDtypeStruct((d, d), f32),
    )(vts, v)

    return (gradient, matrix)


# speedup vs baseline: 2.4441x; 1.5616x over previous
import jax
import jax.numpy as jnp
from jax.experimental import pallas as pl
from jax.experimental.pallas import tpu as pltpu


D = 4096
BLK = 128
CHUNK = 256
NB = D // BLK
ITERS = 4
G_A = NB
G_B = G_A + ITERS
G_TOT = G_B + NB
EPS = 1e-6


def _normalized(x):
    return x * jax.lax.rsqrt(jnp.sum(x * x))


def _mega_kernel(a_ref, g_ref, m_ref, mb, col_a, cs_row, v_row, vt_row):
    g = pl.program_id(0)

    @pl.when(g < G_A)
    def _phase_a():
        a = a_ref[...]
        sq = a * a
        m_ref[...] = sq
        base = pl.multiple_of(g * BLK, BLK)
        mb[pl.ds(base, BLK), :] = sq.astype(jnp.bfloat16)
        col_a[pl.ds(base, BLK), :] = jnp.sum(sq, axis=1, keepdims=True)
        part = jnp.sum(sq, axis=0, keepdims=True)

        @pl.when(g == 0)
        def _init():
            cs_row[...] = part

        @pl.when(g != 0)
        def _acc():
            cs_row[...] += part

    @pl.when(g == G_A)
    def _iter_init():
        eps_d = jnp.float32(EPS) * D
        v_row[...] = _normalized(col_a[...].reshape(1, D) + eps_d)
        vt_row[...] = _normalized(cs_row[...] + eps_d)

    @pl.when((g >= G_A) & (g < G_B))
    def _phase_b():
        vrow = v_row[...]
        vtrow = vt_row[...]
        sv = jnp.sum(vrow) * jnp.float32(EPS)
        svt = jnp.sum(vtrow) * jnp.float32(EPS)
        vb = vrow.astype(jnp.bfloat16)
        vtb = vtrow.astype(jnp.bfloat16)
        y_parts = []
        yt = jnp.zeros((1, D), jnp.float32)
        for c in range(0, D, CHUNK):
            m_chunk = mb[c:c + CHUNK, :]
            y_parts.append(jax.lax.dot_general(
                vb, m_chunk, (((1,), (1,)), ((), ())),
                preferred_element_type=jnp.float32))
            yt = yt + jax.lax.dot_general(
                vtb[:, c:c + CHUNK], m_chunk, (((1,), (0,)), ((), ())),
                preferred_element_type=jnp.float32)
        y = jnp.concatenate(y_parts, axis=1)
        v_row[...] = _normalized(y + sv)
        vt_row[...] = _normalized(yt + svt)

    @pl.when(g >= G_B)
    def _phase_c():
        t = g - G_B

        @pl.when(g == G_B)
        def _scale():
            inv_dot = jnp.float32(1.0) / jnp.sum(v_row[...] * vt_row[...])
            col_a[...] = (vt_row[...] * inv_dot).reshape(D, 1)

        vts_blk = col_a[pl.ds(pl.multiple_of(t * BLK, BLK), BLK), :]
        out = vts_blk * v_row[...]
        rows = jax.lax.broadcasted_iota(jnp.int32, (BLK, D), 0) + t * BLK
        cols = jax.lax.broadcasted_iota(jnp.int32, (BLK, D), 1)
        g_ref[...] = out + jnp.where(rows == cols, jnp.float32(100.0),
                                     jnp.float32(0.0))


def kernel(adj_mtx):
    f32 = jnp.float32
    last = NB - 1

    gradient, matrix = pl.pallas_call(
        _mega_kernel,
        grid=(G_TOT,),
        in_specs=[
            pl.BlockSpec((BLK, D), lambda i: (jnp.minimum(i, last), 0)),
        ],
        out_specs=[
            pl.BlockSpec((BLK, D), lambda i: (jnp.maximum(i - G_B, 0), 0)),
            pl.BlockSpec((BLK, D), lambda i: (jnp.minimum(i, last), 0)),
        ],
        out_shape=[
            jax.ShapeDtypeStruct((D, D), f32),
            jax.ShapeDtypeStruct((D, D), f32),
        ],
        scratch_shapes=[
            pltpu.VMEM((D, D), jnp.bfloat16),
            pltpu.VMEM((D, 1), f32),
            pltpu.VMEM((1, D), f32),
            pltpu.VMEM((1, D), f32),
            pltpu.VMEM((1, D), f32),
        ],
    )(adj_mtx)

    return (gradient, matrix)


# speedup vs baseline: 2.6898x; 1.1005x over previous
import jax
import jax.numpy as jnp
from jax.experimental import pallas as pl
from jax.experimental.pallas import tpu as pltpu


D = 4096
BLK = 256
CHUNK = 256
GBLK = 128
NB = D // BLK
NGB = D // GBLK
ITERS = 4
G_A = NB
G_B = G_A + ITERS
G_TOT = G_B + NGB
EPS = 1e-6


def _normalized(x):
    return x * jax.lax.rsqrt(jnp.sum(x * x))


def _mega_kernel(a_ref, g_ref, m_ref, mb, col_a, cs_row, v_row, vt_row):
    g = pl.program_id(0)

    @pl.when(g < G_A)
    def _phase_a():
        a = a_ref[...]
        sq = a * a
        m_ref[...] = sq
        base = pl.multiple_of(g * BLK, BLK)
        mb[pl.ds(base, BLK), :] = sq.astype(jnp.bfloat16)
        col_a[pl.ds(base, BLK), :] = jnp.sum(sq, axis=1, keepdims=True)
        part = jnp.sum(sq, axis=0, keepdims=True)

        @pl.when(g == 0)
        def _init():
            cs_row[...] = part

        @pl.when(g != 0)
        def _acc():
            cs_row[...] += part

    @pl.when(g == G_A)
    def _iter_init():
        eps_d = jnp.float32(EPS) * D
        v_row[...] = _normalized(col_a[...].reshape(1, D) + eps_d)
        vt_row[...] = _normalized(cs_row[...] + eps_d)

    @pl.when((g >= G_A) & (g < G_B))
    def _phase_b():
        vrow = v_row[...]
        vtrow = vt_row[...]
        sv = jnp.sum(vrow) * jnp.float32(EPS)
        svt = jnp.sum(vtrow) * jnp.float32(EPS)
        vb = vrow.astype(jnp.bfloat16)
        vtb = vtrow.astype(jnp.bfloat16)
        y_parts = []
        yt = jnp.zeros((1, D), jnp.float32)
        for c in range(0, D, CHUNK):
            m_chunk = mb[c:c + CHUNK, :]
            y_parts.append(jax.lax.dot_general(
                vb, m_chunk, (((1,), (1,)), ((), ())),
                preferred_element_type=jnp.float32))
            yt = yt + jax.lax.dot_general(
                vtb[:, c:c + CHUNK], m_chunk, (((1,), (0,)), ((), ())),
                preferred_element_type=jnp.float32)
        y = jnp.concatenate(y_parts, axis=1)
        v_row[...] = _normalized(y + sv)
        vt_row[...] = _normalized(yt + svt)

    @pl.when(g >= G_B)
    def _phase_c():
        t = g - G_B

        @pl.when(g == G_B)
        def _scale():
            inv_dot = jnp.float32(1.0) / jnp.sum(v_row[...] * vt_row[...])
            col_a[...] = (vt_row[...] * inv_dot).reshape(D, 1)

        vts_blk = col_a[pl.ds(pl.multiple_of(t * GBLK, GBLK), GBLK), :]
        g_ref[...] = vts_blk * v_row[...]
        lanes = pl.ds(pl.multiple_of(t * GBLK, GBLK), GBLK)
        r_ids = jax.lax.broadcasted_iota(jnp.int32, (GBLK, GBLK), 0)
        c_ids = jax.lax.broadcasted_iota(jnp.int32, (GBLK, GBLK), 1)
        eye = jnp.where(r_ids == c_ids, jnp.float32(100.0), jnp.float32(0.0))
        g_ref[:, lanes] += eye


def kernel(adj_mtx):
    f32 = jnp.float32
    last = NB - 1

    gradient, matrix = pl.pallas_call(
        _mega_kernel,
        grid=(G_TOT,),
        compiler_params=pltpu.CompilerParams(
            vmem_limit_bytes=64 * 1024 * 1024),
        in_specs=[
            pl.BlockSpec((BLK, D), lambda i: (jnp.minimum(i, last), 0)),
        ],
        out_specs=[
            pl.BlockSpec((GBLK, D), lambda i: (jnp.maximum(i - G_B, 0), 0)),
            pl.BlockSpec((BLK, D), lambda i: (jnp.minimum(i, last), 0)),
        ],
        out_shape=[
            jax.ShapeDtypeStruct((D, D), f32),
            jax.ShapeDtypeStruct((D, D), f32),
        ],
        scratch_shapes=[
            pltpu.VMEM((D, D), jnp.bfloat16),
            pltpu.VMEM((D, 1), f32),
            pltpu.VMEM((1, D), f32),
            pltpu.VMEM((1, D), f32),
            pltpu.VMEM((1, D), f32),
        ],
    )(adj_mtx)

    return (gradient, matrix)


# speedup vs baseline: 3.5157x; 1.3070x over previous
import jax
import jax.numpy as jnp
from jax.experimental import pallas as pl
from jax.experimental.pallas import tpu as pltpu


D = 4096
BLK = 256
CHUNK = 256
GBLK = 256
NB = D // BLK
NGB = D // GBLK
ITERS = 4
G_A = NB
G_B = G_A + ITERS
G_TOT = G_B + NGB
EPS = 1e-6


def _normalized(x):
    return x * jax.lax.rsqrt(jnp.sum(x * x))


def _mega_kernel(a_ref, g_ref, m_ref, mb, col_a, cs_row, v_row, vt_row):
    g = pl.program_id(0)

    @pl.when(g < G_A)
    def _phase_a():
        a = a_ref[...]
        sq = a * a
        m_ref[...] = sq
        base = pl.multiple_of(g * BLK, BLK)
        mb[pl.ds(base, BLK), :] = sq.astype(jnp.float8_e4m3fn)
        col_a[pl.ds(base, BLK), :] = jnp.sum(sq, axis=1, keepdims=True)
        part = jnp.sum(sq, axis=0, keepdims=True)

        @pl.when(g == 0)
        def _init():
            cs_row[...] = part

        @pl.when(g != 0)
        def _acc():
            cs_row[...] += part

    @pl.when(g == G_A)
    def _iter_init():
        eps_d = jnp.float32(EPS) * D
        v_row[...] = _normalized(col_a[...].reshape(1, D) + eps_d)
        vt_row[...] = _normalized(cs_row[...] + eps_d)

    @pl.when((g >= G_A) & (g < G_B))
    def _phase_b():
        vrow = v_row[...]
        vtrow = vt_row[...]
        sv = jnp.sum(vrow) * jnp.float32(EPS)
        svt = jnp.sum(vtrow) * jnp.float32(EPS)
        vb = vrow.astype(jnp.float8_e4m3fn)
        vtb = vtrow.astype(jnp.float8_e4m3fn)
        y_parts = []
        yt = jnp.zeros((1, D), jnp.float32)
        for c in range(0, D, CHUNK):
            m_chunk = mb[c:c + CHUNK, :]
            y_parts.append(jax.lax.dot_general(
                vb, m_chunk, (((1,), (1,)), ((), ())),
                preferred_element_type=jnp.float32))
            yt = yt + jax.lax.dot_general(
                vtb[:, c:c + CHUNK], m_chunk, (((1,), (0,)), ((), ())),
                preferred_element_type=jnp.float32)
        y = jnp.concatenate(y_parts, axis=1)
        v_row[...] = _normalized(y + sv)
        vt_row[...] = _normalized(yt + svt)

    @pl.when(g >= G_B)
    def _phase_c():
        t = g - G_B

        @pl.when(g == G_B)
        def _scale():
            inv_dot = jnp.float32(1.0) / jnp.sum(v_row[...] * vt_row[...])
            col_a[...] = (vt_row[...] * inv_dot).reshape(D, 1)

        vts_blk = col_a[pl.ds(pl.multiple_of(t * GBLK, GBLK), GBLK), :]
        g_ref[...] = vts_blk * v_row[...]
        lanes = pl.ds(pl.multiple_of(t * GBLK, GBLK), GBLK)
        r_ids = jax.lax.broadcasted_iota(jnp.int32, (GBLK, GBLK), 0)
        c_ids = jax.lax.broadcasted_iota(jnp.int32, (GBLK, GBLK), 1)
        eye = jnp.where(r_ids == c_ids, jnp.float32(100.0), jnp.float32(0.0))
        g_ref[:, lanes] += eye


def kernel(adj_mtx):
    f32 = jnp.float32
    last = NB - 1

    gradient, matrix = pl.pallas_call(
        _mega_kernel,
        grid=(G_TOT,),
        compiler_params=pltpu.CompilerParams(
            vmem_limit_bytes=64 * 1024 * 1024),
        in_specs=[
            pl.BlockSpec((BLK, D), lambda i: (jnp.minimum(i, last), 0)),
        ],
        out_specs=[
            pl.BlockSpec((GBLK, D), lambda i: (jnp.maximum(i - G_B, 0), 0)),
            pl.BlockSpec((BLK, D), lambda i: (jnp.minimum(i, last), 0)),
        ],
        out_shape=[
            jax.ShapeDtypeStruct((D, D), f32),
            jax.ShapeDtypeStruct((D, D), f32),
        ],
        scratch_shapes=[
            pltpu.VMEM((D, D), jnp.float8_e4m3fn),
            pltpu.VMEM((D, 1), f32),
            pltpu.VMEM((1, D), f32),
            pltpu.VMEM((1, D), f32),
            pltpu.VMEM((1, D), f32),
        ],
    )(adj_mtx)

    return (gradient, matrix)


# speedup vs baseline: 3.6782x; 1.0462x over previous
import jax
import jax.numpy as jnp
from jax.experimental import pallas as pl
from jax.experimental.pallas import tpu as pltpu


D = 4096
BLK = 512
CHUNK = 256
GBLK = 256
NB = D // BLK
NGB = D // GBLK
ITERS = 4
G_A = NB
G_B = G_A + ITERS
G_TOT = G_B + NGB
EPS = 1e-6


def _normalized(x):
    return x * jax.lax.rsqrt(jnp.sum(x * x))


def _mega_kernel(a_ref, g_ref, m_ref, mb, rs_row, cs_row, v_row, vt_row):
    g = pl.program_id(0)

    @pl.when(g < G_A)
    def _phase_a():
        a = a_ref[...]
        sq = a * a
        m_ref[...] = sq
        base = pl.multiple_of(g * BLK, BLK)
        mb[pl.ds(base, BLK), :] = sq.astype(jnp.float8_e4m3fn)
        rs_row[:, pl.ds(base, BLK)] = (
            jnp.sum(sq, axis=1, keepdims=True).reshape(1, BLK))
        part = jnp.sum(sq, axis=0, keepdims=True)

        @pl.when(g == 0)
        def _init():
            cs_row[...] = part

        @pl.when(g != 0)
        def _acc():
            cs_row[...] += part

    @pl.when(g == G_A)
    def _iter_init():
        eps_d = jnp.float32(EPS) * D
        v_row[...] = _normalized(rs_row[...] + eps_d)
        vt_row[...] = _normalized(cs_row[...] + eps_d)

    @pl.when((g >= G_A) & (g < G_B))
    def _phase_b():
        vrow = v_row[...]
        vtrow = vt_row[...]
        sv = jnp.sum(vrow) * jnp.float32(EPS)
        svt = jnp.sum(vtrow) * jnp.float32(EPS)
        vb = vrow.astype(jnp.float8_e4m3fn)
        vtb = vtrow.astype(jnp.float8_e4m3fn)
        y_parts = []
        yt = jnp.zeros((1, D), jnp.float32)
        for c in range(0, D, CHUNK):
            m_chunk = mb[c:c + CHUNK, :]
            y_parts.append(jax.lax.dot_general(
                vb, m_chunk, (((1,), (1,)), ((), ())),
                preferred_element_type=jnp.float32))
            yt = yt + jax.lax.dot_general(
                vtb[:, c:c + CHUNK], m_chunk, (((1,), (0,)), ((), ())),
                preferred_element_type=jnp.float32)
        y = jnp.concatenate(y_parts, axis=1)
        v_row[...] = _normalized(y + sv)
        vt_row[...] = _normalized(yt + svt)

    @pl.when(g >= G_B)
    def _phase_c():
        t = g - G_B

        @pl.when(g == G_B)
        def _scale():
            inv_dot = jnp.float32(1.0) / jnp.sum(v_row[...] * vt_row[...])
            rs_row[...] = vt_row[...] * inv_dot

        vts_blk = rs_row[:, pl.ds(pl.multiple_of(t * GBLK, GBLK), GBLK)]
        g_ref[...] = vts_blk.reshape(GBLK, 1) * v_row[...]
        lanes = pl.ds(pl.multiple_of(t * GBLK, GBLK), GBLK)
        r_ids = jax.lax.broadcasted_iota(jnp.int32, (GBLK, GBLK), 0)
        c_ids = jax.lax.broadcasted_iota(jnp.int32, (GBLK, GBLK), 1)
        eye = jnp.where(r_ids == c_ids, jnp.float32(100.0), jnp.float32(0.0))
        g_ref[:, lanes] += eye


def kernel(adj_mtx):
    f32 = jnp.float32
    last = NB - 1

    gradient, matrix = pl.pallas_call(
        _mega_kernel,
        grid=(G_TOT,),
        compiler_params=pltpu.CompilerParams(
            vmem_limit_bytes=64 * 1024 * 1024),
        in_specs=[
            pl.BlockSpec((BLK, D), lambda i: (jnp.minimum(i, last), 0)),
        ],
        out_specs=[
            pl.BlockSpec((GBLK, D), lambda i: (jnp.maximum(i - G_B, 0), 0)),
            pl.BlockSpec((BLK, D), lambda i: (jnp.minimum(i, last), 0)),
        ],
        out_shape=[
            jax.ShapeDtypeStruct((D, D), f32),
            jax.ShapeDtypeStruct((D, D), f32),
        ],
        scratch_shapes=[
            pltpu.VMEM((D, D), jnp.float8_e4m3fn),
            pltpu.VMEM((1, D), f32),
            pltpu.VMEM((1, D), f32),
            pltpu.VMEM((1, D), f32),
            pltpu.VMEM((1, D), f32),
        ],
    )(adj_mtx)

    return (gradient, matrix)
